# Initial kernel scaffold; baseline (speedup 1.0000x reference)
import jax
import jax.numpy as jnp
from jax.experimental import pallas as pl


def kernel(memory, read_keys, beta, W_merge, b_merge, ln_gamma, ln_beta):
    raise NotImplementedError("write your pallas kernel here")


# speedup vs baseline: 2.6251x; 2.6251x over previous
import functools

import jax
import jax.numpy as jnp
import numpy as np
from jax.experimental import pallas as pl

_TOPK = 64


def _sim_topk_body(mem_ref, keys_ref, beta_ref, w_ref, *, chunk, n_chunks, n, k):
    nc = pl.program_id(1)
    kk = keys_ref[0]
    kn = kk / jnp.maximum(jnp.sqrt(jnp.sum(kk * kk, axis=1, keepdims=True)), 1e-12)
    m = mem_ref[0]
    mn = m / jnp.maximum(jnp.sqrt(jnp.sum(m * m, axis=1, keepdims=True)), 1e-12)
    sim = jax.lax.dot_general(kn, mn, (((1,), (1,)), ((), ())),
                              preferred_element_type=jnp.float32)
    sim = sim * beta_ref[0, 0][:, None]
    w_ref[0, :, pl.ds(nc * chunk, chunk)] = sim

    @pl.when(nc == n_chunks - 1)
    def _():
        h = w_ref.shape[1]
        s0 = w_ref[0]
        s = jnp.where(s0 == 0.0, jnp.float32(0.0), s0)
        bits = jax.lax.bitcast_convert_type(s, jnp.int32)
        keys = jnp.where(bits < 0, bits ^ jnp.int32(0x7FFFFFFF), bits)
        sign = jnp.int32(-(2 ** 31))
        u = jnp.zeros((h, 1), jnp.int32)
        for i in range(31, -1, -1):
            bit = jnp.int32(np.int32(np.uint32(1 << i)))
            cand = u | bit
            ck = cand ^ sign
            cnt = jnp.sum((keys >= ck).astype(jnp.int32), axis=1, keepdims=True)
            u = jnp.where(cnt >= k, cand, u)
        t = u ^ sign
        gt = keys > t
        eq = keys == t
        cnt_gt = jnp.sum(gt.astype(jnp.int32), axis=1, keepdims=True)
        need = k - cnt_gt
        iota = jax.lax.broadcasted_iota(jnp.int32, (h, n), 1)
        j0 = jnp.zeros((h, 1), jnp.int32)
        for i in range(12, -1, -1):
            candj = j0 | jnp.int32(1 << i)
            c = jnp.sum((eq & (iota < candj)).astype(jnp.int32), axis=1,
                        keepdims=True)
            j0 = jnp.where(c < need, candj, j0)
        sel = gt | (eq & (iota <= j0))
        mx = jnp.max(s, axis=1, keepdims=True)
        e = jnp.where(sel, jnp.exp(s - mx), 0.0)
        w_ref[0] = e / jnp.sum(e, axis=1, keepdims=True)


def _read_body(w_ref, mem_ref, out_ref):
    nc = pl.program_id(1)

    @pl.when(nc == 0)
    def _():
        out_ref[...] = jnp.zeros_like(out_ref)

    out_ref[0] += jax.lax.dot_general(
        w_ref[0], mem_ref[0], (((1,), (0,)), ((), ())),
        preferred_element_type=jnp.float32)


def _merge_body(flat_ref, wm_ref, bm_ref, g_ref, lb_ref, out_ref):
    merged = jax.lax.dot_general(flat_ref[...], wm_ref[...],
                                 (((1,), (1,)), ((), ())),
                                 preferred_element_type=jnp.float32)
    merged = merged + bm_ref[...]
    mu = jnp.mean(merged, axis=-1, keepdims=True)
    var = jnp.mean((merged - mu) ** 2, axis=-1, keepdims=True)
    out_ref[...] = (merged - mu) / jnp.sqrt(var + 1e-5) * g_ref[...] + lb_ref[...]


def kernel(memory, read_keys, beta, W_merge, b_merge, ln_gamma, ln_beta):
    B, N, D = memory.shape
    H = read_keys.shape[1]
    k = min(_TOPK, N)
    chunk = min(2048, N)
    n_chunks = N // chunk

    weights = pl.pallas_call(
        functools.partial(_sim_topk_body, chunk=chunk, n_chunks=n_chunks,
                          n=N, k=k),
        grid=(B, n_chunks),
        in_specs=[
            pl.BlockSpec((1, chunk, D), lambda b, nc: (b, nc, 0)),
            pl.BlockSpec((1, H, D), lambda b, nc: (b, 0, 0)),
            pl.BlockSpec((1, 1, H), lambda b, nc: (b, 0, 0)),
        ],
        out_specs=pl.BlockSpec((1, H, N), lambda b, nc: (b, 0, 0)),
        out_shape=jax.ShapeDtypeStruct((B, H, N), jnp.float32),
    )(memory, read_keys, beta.reshape(B, 1, H))

    read_per_head = pl.pallas_call(
        _read_body,
        grid=(B, n_chunks),
        in_specs=[
            pl.BlockSpec((1, H, chunk), lambda b, nc: (b, 0, nc)),
            pl.BlockSpec((1, chunk, D), lambda b, nc: (b, nc, 0)),
        ],
        out_specs=pl.BlockSpec((1, H, D), lambda b, nc: (b, 0, 0)),
        out_shape=jax.ShapeDtypeStruct((B, H, D), jnp.float32),
    )(weights, memory)

    flat = read_per_head.reshape(B, H * D)
    read_combined = pl.pallas_call(
        _merge_body,
        in_specs=[
            pl.BlockSpec((B, H * D), lambda: (0, 0)),
            pl.BlockSpec((D, H * D), lambda: (0, 0)),
            pl.BlockSpec((1, D), lambda: (0, 0)),
            pl.BlockSpec((1, D), lambda: (0, 0)),
            pl.BlockSpec((1, D), lambda: (0, 0)),
        ],
        out_specs=pl.BlockSpec((B, D), lambda: (0, 0)),
        out_shape=jax.ShapeDtypeStruct((B, D), jnp.float32),
    )(flat, W_merge, b_merge.reshape(1, D), ln_gamma.reshape(1, D),
      ln_beta.reshape(1, D))

    return read_combined, weights


# speedup vs baseline: 4.0728x; 1.5515x over previous
import functools

import jax
import jax.numpy as jnp
import numpy as np
from jax import lax
from jax.experimental import pallas as pl

_TOPK = 64


def _fused_body(mem_ref, keys_ref, beta_ref, w_ref, rph_ref, *, n, k, chunk):
    h = w_ref.shape[1]
    kk = keys_ref[0]
    kn = kk / jnp.maximum(jnp.sqrt(jnp.sum(kk * kk, axis=1, keepdims=True)),
                          1e-12)
    for c in range(n // chunk):
        m = mem_ref[0, pl.ds(c * chunk, chunk), :]
        mn = m / jnp.maximum(
            jnp.sqrt(jnp.sum(m * m, axis=1, keepdims=True)), 1e-12)
        sim = lax.dot_general(kn, mn, (((1,), (1,)), ((), ())),
                              preferred_element_type=jnp.float32)
        w_ref[0, :, pl.ds(c * chunk, chunk)] = sim * beta_ref[0, 0][:, None]

    s0 = w_ref[0]
    s = jnp.where(s0 == 0.0, jnp.float32(0.0), s0)
    bits = lax.bitcast_convert_type(s, jnp.int32)
    keys = jnp.where(bits < 0, bits ^ jnp.int32(0x7FFFFFFF), bits)
    sign = jnp.int32(-(2 ** 31))
    u = jnp.zeros((h, 1), jnp.int32)
    for i in range(31, -1, -1):
        bit = jnp.int32(np.int32(np.uint32(1 << i)))
        cand = u | bit
        ck = cand ^ sign
        cnt = jnp.sum((keys >= ck).astype(jnp.int32), axis=1, keepdims=True)
        u = jnp.where(cnt >= k, cand, u)
    t = u ^ sign
    gt = keys > t
    eq = keys == t
    cnt_gt = jnp.sum(gt.astype(jnp.int32), axis=1, keepdims=True)
    need = k - cnt_gt
    iota = lax.broadcasted_iota(jnp.int32, (h, n), 1)
    j0 = jnp.zeros((h, 1), jnp.int32)
    for i in range(12, -1, -1):
        candj = j0 | jnp.int32(1 << i)
        c = jnp.sum((eq & (iota < candj)).astype(jnp.int32), axis=1,
                    keepdims=True)
        j0 = jnp.where(c < need, candj, j0)
    sel = gt | (eq & (iota <= j0))
    mx = jnp.max(s, axis=1, keepdims=True)
    e = jnp.where(sel, jnp.exp(s - mx), 0.0)
    w = e / jnp.sum(e, axis=1, keepdims=True)
    w_ref[0] = w
    acc = jnp.zeros((h, rph_ref.shape[2]), jnp.float32)
    for c in range(n // chunk):
        acc = acc + lax.dot_general(
            w[:, c * chunk:(c + 1) * chunk],
            mem_ref[0, pl.ds(c * chunk, chunk), :],
            (((1,), (0,)), ((), ())), preferred_element_type=jnp.float32)
    rph_ref[0] = acc


def _merge_body(flat_ref, wm_ref, bm_ref, g_ref, lb_ref, out_ref):
    merged = lax.dot_general(flat_ref[...], wm_ref[...],
                             (((1,), (1,)), ((), ())),
                             preferred_element_type=jnp.float32)
    merged = merged + bm_ref[...]
    mu = jnp.mean(merged, axis=-1, keepdims=True)
    var = jnp.mean((merged - mu) ** 2, axis=-1, keepdims=True)
    out_ref[...] = (merged - mu) / jnp.sqrt(var + 1e-5) * g_ref[...] + lb_ref[...]


def kernel(memory, read_keys, beta, W_merge, b_merge, ln_gamma, ln_beta):
    B, N, D = memory.shape
    H = read_keys.shape[1]
    k = min(_TOPK, N)
    chunk = min(2048, N)

    weights, read_per_head = pl.pallas_call(
        functools.partial(_fused_body, n=N, k=k, chunk=chunk),
        grid=(B,),
        in_specs=[
            pl.BlockSpec((1, N, D), lambda b: (b, 0, 0)),
            pl.BlockSpec((1, H, D), lambda b: (b, 0, 0)),
            pl.BlockSpec((1, 1, H), lambda b: (b, 0, 0)),
        ],
        out_specs=[
            pl.BlockSpec((1, H, N), lambda b: (b, 0, 0)),
            pl.BlockSpec((1, H, D), lambda b: (b, 0, 0)),
        ],
        out_shape=[
            jax.ShapeDtypeStruct((B, H, N), jnp.float32),
            jax.ShapeDtypeStruct((B, H, D), jnp.float32),
        ],
    )(memory, read_keys, beta.reshape(B, 1, H))

    flat = read_per_head.reshape(B, H * D)
    read_combined = pl.pallas_call(
        _merge_body,
        in_specs=[
            pl.BlockSpec((B, H * D), lambda: (0, 0)),
            pl.BlockSpec((D, H * D), lambda: (0, 0)),
            pl.BlockSpec((1, D), lambda: (0, 0)),
            pl.BlockSpec((1, D), lambda: (0, 0)),
            pl.BlockSpec((1, D), lambda: (0, 0)),
        ],
        out_specs=pl.BlockSpec((B, D), lambda: (0, 0)),
        out_shape=jax.ShapeDtypeStruct((B, D), jnp.float32),
    )(flat, W_merge, b_merge.reshape(1, D), ln_gamma.reshape(1, D),
      ln_beta.reshape(1, D))

    return read_combined, weights


# speedup vs baseline: 4.7613x; 1.1690x over previous
import functools

import jax
import jax.numpy as jnp
import numpy as np
from jax import lax
from jax.experimental import pallas as pl

_TOPK = 64


def _fused_body(mem_ref, keys_ref, beta_ref, w_ref, rph_ref, *, n, k, chunk):
    h = w_ref.shape[1]
    kk = keys_ref[0]
    kn = kk / jnp.maximum(jnp.sqrt(jnp.sum(kk * kk, axis=1, keepdims=True)),
                          1e-12)
    for c in range(n // chunk):
        m = mem_ref[0, pl.ds(c * chunk, chunk), :]
        mn = m / jnp.maximum(
            jnp.sqrt(jnp.sum(m * m, axis=1, keepdims=True)), 1e-12)
        sim = lax.dot_general(kn, mn, (((1,), (1,)), ((), ())),
                              preferred_element_type=jnp.float32)
        w_ref[0, :, pl.ds(c * chunk, chunk)] = sim * beta_ref[0, 0][:, None]

    s0 = w_ref[0]
    s = jnp.where(s0 == 0.0, jnp.float32(0.0), s0)
    bits = lax.bitcast_convert_type(s, jnp.int32)
    keys = jnp.where(bits < 0, bits ^ jnp.int32(0x7FFFFFFF), bits)
    sign = jnp.int32(-(2 ** 31))
    u = jnp.zeros((h, 1), jnp.int32)
    for i in range(31, -1, -1):
        bit = jnp.int32(np.int32(np.uint32(1 << i)))
        cand = u | bit
        ck = cand ^ sign
        cnt = jnp.sum((keys >= ck).astype(jnp.int32), axis=1, keepdims=True)
        u = jnp.where(cnt >= k, cand, u)
    t = u ^ sign
    gt = keys > t
    eq = keys == t
    cnt_gt = jnp.sum(gt.astype(jnp.int32), axis=1, keepdims=True)
    need = k - cnt_gt
    cnt_eq = jnp.sum(eq.astype(jnp.int32), axis=1, keepdims=True)
    iota = lax.broadcasted_iota(jnp.int32, (h, n), 1)

    def _tie_break(_):
        j0 = jnp.zeros((h, 1), jnp.int32)
        for i in range(12, -1, -1):
            candj = j0 | jnp.int32(1 << i)
            c = jnp.sum((eq & (iota < candj)).astype(jnp.int32), axis=1,
                        keepdims=True)
            j0 = jnp.where(c < need, candj, j0)
        return j0

    j0 = lax.cond(jnp.any(cnt_eq > need), _tie_break,
                  lambda _: jnp.full((h, 1), jnp.int32(n)), None)
    sel = gt | (eq & (iota <= j0))
    mx = jnp.max(s, axis=1, keepdims=True)
    e = jnp.where(sel, jnp.exp(s - mx), 0.0)
    w = e / jnp.sum(e, axis=1, keepdims=True)
    w_ref[0] = w
    acc = jnp.zeros((h, rph_ref.shape[2]), jnp.float32)
    for c in range(n // chunk):
        acc = acc + lax.dot_general(
            w[:, c * chunk:(c + 1) * chunk],
            mem_ref[0, pl.ds(c * chunk, chunk), :],
            (((1,), (0,)), ((), ())), preferred_element_type=jnp.float32)
    rph_ref[0] = acc


def _merge_body(flat_ref, wm_ref, bm_ref, g_ref, lb_ref, out_ref):
    merged = lax.dot_general(flat_ref[...], wm_ref[...],
                             (((1,), (1,)), ((), ())),
                             preferred_element_type=jnp.float32)
    merged = merged + bm_ref[...]
    mu = jnp.mean(merged, axis=-1, keepdims=True)
    var = jnp.mean((merged - mu) ** 2, axis=-1, keepdims=True)
    out_ref[...] = (merged - mu) / jnp.sqrt(var + 1e-5) * g_ref[...] + lb_ref[...]


def kernel(memory, read_keys, beta, W_merge, b_merge, ln_gamma, ln_beta):
    B, N, D = memory.shape
    H = read_keys.shape[1]
    k = min(_TOPK, N)
    chunk = min(2048, N)

    weights, read_per_head = pl.pallas_call(
        functools.partial(_fused_body, n=N, k=k, chunk=chunk),
        grid=(B,),
        in_specs=[
            pl.BlockSpec((1, N, D), lambda b: (b, 0, 0)),
            pl.BlockSpec((1, H, D), lambda b: (b, 0, 0)),
            pl.BlockSpec((1, 1, H), lambda b: (b, 0, 0)),
        ],
        out_specs=[
            pl.BlockSpec((1, H, N), lambda b: (b, 0, 0)),
            pl.BlockSpec((1, H, D), lambda b: (b, 0, 0)),
        ],
        out_shape=[
            jax.ShapeDtypeStruct((B, H, N), jnp.float32),
            jax.ShapeDtypeStruct((B, H, D), jnp.float32),
        ],
    )(memory, read_keys, beta.reshape(B, 1, H))

    flat = read_per_head.reshape(B, H * D)
    read_combined = pl.pallas_call(
        _merge_body,
        in_specs=[
            pl.BlockSpec((B, H * D), lambda: (0, 0)),
            pl.BlockSpec((D, H * D), lambda: (0, 0)),
            pl.BlockSpec((1, D), lambda: (0, 0)),
            pl.BlockSpec((1, D), lambda: (0, 0)),
            pl.BlockSpec((1, D), lambda: (0, 0)),
        ],
        out_specs=pl.BlockSpec((B, D), lambda: (0, 0)),
        out_shape=jax.ShapeDtypeStruct((B, D), jnp.float32),
    )(flat, W_merge, b_merge.reshape(1, D), ln_gamma.reshape(1, D),
      ln_beta.reshape(1, D))

    return read_combined, weights
